# baseline (speedup 1.0000x reference)
import functools

import jax
import jax.numpy as jnp
from jax import lax
from jax.experimental import pallas as pl
from jax.experimental.pallas import tpu as pltpu
from jax.experimental.pallas import tpu_sc as plsc

N = 160000
B = 4096
H = 128
L = 128
VC = 133
VI = 495

_NW = 32
_CHUNK = 40
_CPW = N // (_NW * _CHUNK)
_D = L

_NB = 1000

_BF = jnp.bfloat16
_F32 = jnp.float32


def _sc_gather(table, idx3):
    mesh = plsc.VectorSubcoreMesh(core_axis_name="c", subcore_axis_name="s")

    @functools.partial(
        pl.kernel,
        mesh=mesh,
        out_type=jax.ShapeDtypeStruct((N, _D), jnp.int32),
        scratch_types=[
            pltpu.VMEM((_CPW, _CHUNK), jnp.int32),
            pltpu.VMEM((_CHUNK, _D), jnp.int32),
            pltpu.VMEM((_CHUNK, _D), jnp.int32),
            pltpu.SemaphoreType.DMA,
            pltpu.SemaphoreType.DMA,
        ],
    )
    def gather_kernel(table_hbm, idx_hbm, out_hbm, idx_v, buf0, buf1, sem0, sem1):
        wid = lax.axis_index("s") * 2 + lax.axis_index("c")
        cbase = wid * _CPW
        pltpu.sync_copy(idx_hbm.at[wid], idx_v)

        def gat(c, buf, sem):
            return pltpu.make_async_copy(table_hbm.at[idx_v.at[c]], buf, sem)

        def st(c, buf):
            pltpu.sync_copy(buf, out_hbm.at[pl.ds((cbase + c) * _CHUNK, _CHUNK)])

        gat(0, buf0, sem0).start()

        def body(i, carry):
            c0 = 2 * i
            gat(c0, buf0, sem0).wait()
            gat(c0 + 1, buf1, sem1).start()
            st(c0, buf0)
            gat(c0 + 1, buf1, sem1).wait()
            gat(c0 + 2, buf0, sem0).start()
            st(c0 + 1, buf1)
            return carry

        lax.fori_loop(0, (_CPW - 1) // 2, body, 0)
        gat(_CPW - 1, buf0, sem0).wait()
        st(_CPW - 1, buf0)

    return gather_kernel(table, idx3)


def _tc_body(ctx_ref, topo_ref, cls_ref, assm_ref,
             W1t_ref, b1t_ref, w2t_ref, b2t_ref,
             W1ci_ref, b1ci_ref, W2c_ref, b2c_ref, W2i_ref, b2i_ref,
             Wa_ref, ba_ref,
             topo_out, cls_out, icls_out, assm_out):
    tree = ctx_ref[:, :L]
    graph = ctx_ref[:, L:]
    h_t = jnp.dot(topo_ref[...], W1t_ref[:H], preferred_element_type=_F32)
    h_t = h_t + jnp.dot(tree, W1t_ref[H:], preferred_element_type=_F32)
    h_t = jnp.maximum(h_t + b1t_ref[...], 0.0)
    topo_out[...] = (jnp.sum(h_t * w2t_ref[...], axis=1, keepdims=True)
                     + b2t_ref[...])
    h_ci = jnp.dot(cls_ref[...], W1ci_ref[:H], preferred_element_type=_F32)
    h_ci = h_ci + jnp.dot(tree, W1ci_ref[H:], preferred_element_type=_F32)
    h_ci = jnp.maximum(h_ci + b1ci_ref[...], 0.0).astype(_BF)
    cls_out[...] = (jnp.dot(h_ci[:, :H], W2c_ref[...], preferred_element_type=_F32)
                    + b2c_ref[...])
    icls_out[...] = (jnp.dot(h_ci[:, H:], W2i_ref[...], preferred_element_type=_F32)
                     + b2i_ref[...])
    a = jnp.dot(assm_ref[...], Wa_ref[...], preferred_element_type=_F32) + ba_ref[...]
    assm_out[...] = jnp.sum(a * graph.astype(_F32), axis=1, keepdims=True)


def _tc_main(ctx, topo_vecs, cls_vecs, assm_vecs,
             W1t, b1t, w2t, b2t, W1ci, b1ci, W2c, b2c, W2i, b2i, Wa, ba):
    grid = (N // _NB,)
    row = lambda w: pl.BlockSpec((_NB, w), lambda i: (i, 0))
    full = lambda a: pl.BlockSpec(a.shape, lambda i: (0,) * a.ndim)
    return pl.pallas_call(
        _tc_body,
        grid=grid,
        in_specs=[
            row(2 * L), row(H), row(H), row(H),
            full(W1t), full(b1t), full(w2t), full(b2t),
            full(W1ci), full(b1ci), full(W2c), full(b2c), full(W2i), full(b2i),
            full(Wa), full(ba),
        ],
        out_specs=[row(1), row(VC), row(VI), row(1)],
        out_shape=[
            jax.ShapeDtypeStruct((N, 1), _F32),
            jax.ShapeDtypeStruct((N, VC), _F32),
            jax.ShapeDtypeStruct((N, VI), _F32),
            jax.ShapeDtypeStruct((N, 1), _F32),
        ],
        compiler_params=pltpu.CompilerParams(
            dimension_semantics=("parallel",),
        ),
    )(ctx, topo_vecs, cls_vecs, assm_vecs,
      W1t, b1t, w2t, b2t, W1ci, b1ci, W2c, b2c, W2i, b2i, Wa, ba)


def kernel(src_tree_vecs, src_graph_vecs, topo_vecs, cls_vecs, assm_vecs,
           batch_idx,
           W1_topo, b1_topo, W2_topo, b2_topo,
           W1_cls, b1_cls, W2_cls, b2_cls,
           W1_icls, b1_icls, W2_icls, b2_icls,
           W_assm, b_assm):
    table_bf = jnp.concatenate([src_tree_vecs, src_graph_vecs], axis=1).astype(_BF)
    table_i32 = lax.bitcast_convert_type(table_bf.reshape(B, 2 * L // 2, 2),
                                         jnp.int32)
    idx3 = batch_idx.reshape(_NW, _CPW, _CHUNK)
    ctx_i32 = _sc_gather(table_i32, idx3)
    ctx = lax.bitcast_convert_type(ctx_i32, _BF).reshape(N, 2 * L)

    W1ci = jnp.concatenate([W1_cls, W1_icls], axis=1)
    b1ci = jnp.concatenate([b1_cls, b1_icls]).reshape(1, 2 * H)
    topo2, cls_s, icls_s, assm2 = _tc_main(
        ctx, topo_vecs.astype(_BF), cls_vecs.astype(_BF), assm_vecs.astype(_BF),
        W1_topo.astype(_BF), b1_topo.reshape(1, H),
        W2_topo.reshape(1, H), b2_topo.reshape(1, 1),
        W1ci.astype(_BF), b1ci,
        W2_cls.astype(_BF), b2_cls.reshape(1, VC),
        W2_icls.astype(_BF), b2_icls.reshape(1, VI),
        W_assm.astype(_BF), b_assm.reshape(1, L),
    )
    return (topo2[:, 0], cls_s, icls_s, assm2[:, 0])


# speedup vs baseline: 1.5651x; 1.5651x over previous
import functools

import jax
import jax.numpy as jnp
from jax import lax
from jax.experimental import pallas as pl
from jax.experimental.pallas import tpu as pltpu
from jax.experimental.pallas import tpu_sc as plsc

N = 160000
B = 4096
H = 128
L = 128
VC = 133
VI = 495

_NW = 32
_CHUNK = 40
_CPW = N // (_NW * _CHUNK)
_D = 2 * L

_NB = 1000

_BF = jnp.bfloat16
_F32 = jnp.float32


def _sc_gather(table, idx3):
    mesh = plsc.VectorSubcoreMesh(core_axis_name="c", subcore_axis_name="s")

    @functools.partial(
        pl.kernel,
        mesh=mesh,
        out_type=jax.ShapeDtypeStruct((N, _D), _F32),
        scratch_types=[
            pltpu.VMEM((_CPW, _CHUNK), jnp.int32),
            pltpu.VMEM((_CHUNK, _D), _F32),
            pltpu.VMEM((_CHUNK, _D), _F32),
            pltpu.SemaphoreType.DMA,
            pltpu.SemaphoreType.DMA,
        ],
    )
    def gather_kernel(table_hbm, idx_hbm, out_hbm, idx_v, buf0, buf1, sem0, sem1):
        wid = lax.axis_index("s") * 2 + lax.axis_index("c")
        cbase = wid * _CPW
        pltpu.sync_copy(idx_hbm.at[wid], idx_v)

        def gat(c, buf, sem):
            return pltpu.make_async_copy(table_hbm.at[idx_v.at[c]], buf, sem)

        def st(c, buf):
            pltpu.sync_copy(buf, out_hbm.at[pl.ds((cbase + c) * _CHUNK, _CHUNK)])

        gat(0, buf0, sem0).start()

        def body(i, carry):
            c0 = 2 * i
            gat(c0, buf0, sem0).wait()
            gat(c0 + 1, buf1, sem1).start()
            st(c0, buf0)
            gat(c0 + 1, buf1, sem1).wait()
            gat(c0 + 2, buf0, sem0).start()
            st(c0 + 1, buf1)
            return carry

        lax.fori_loop(0, (_CPW - 1) // 2, body, 0)
        gat(_CPW - 1, buf0, sem0).wait()
        st(_CPW - 1, buf0)

    return gather_kernel(table, idx3)


def _tc_body(ctx_ref, topo_ref, cls_ref, assm_ref,
             W1t_ref, b1t_ref, w2t_ref, b2t_ref,
             W1ci_ref, b1ci_ref, W2c_ref, b2c_ref, W2i_ref, b2i_ref,
             Wa_ref, ba_ref,
             topo_out, cls_out, icls_out, assm_out):
    tree = ctx_ref[:, :L].astype(_BF)
    graph = ctx_ref[:, L:]
    h_t = jnp.dot(topo_ref[...], W1t_ref[:H], preferred_element_type=_F32)
    h_t = h_t + jnp.dot(tree, W1t_ref[H:], preferred_element_type=_F32)
    h_t = jnp.maximum(h_t + b1t_ref[...], 0.0)
    topo_out[...] = (jnp.sum(h_t * w2t_ref[...], axis=1, keepdims=True)
                     + b2t_ref[...])
    h_ci = jnp.dot(cls_ref[...], W1ci_ref[:H], preferred_element_type=_F32)
    h_ci = h_ci + jnp.dot(tree, W1ci_ref[H:], preferred_element_type=_F32)
    h_ci = jnp.maximum(h_ci + b1ci_ref[...], 0.0).astype(_BF)
    cls_out[...] = (jnp.dot(h_ci[:, :H], W2c_ref[...], preferred_element_type=_F32)
                    + b2c_ref[...])
    icls_out[...] = (jnp.dot(h_ci[:, H:], W2i_ref[...], preferred_element_type=_F32)
                     + b2i_ref[...])
    a = jnp.dot(assm_ref[...], Wa_ref[...], preferred_element_type=_F32) + ba_ref[...]
    assm_out[...] = jnp.sum(a * graph, axis=1, keepdims=True)


def _tc_main(ctx, topo_vecs, cls_vecs, assm_vecs,
             W1t, b1t, w2t, b2t, W1ci, b1ci, W2c, b2c, W2i, b2i, Wa, ba):
    grid = (N // _NB,)
    row = lambda w: pl.BlockSpec((_NB, w), lambda i: (i, 0))
    full = lambda a: pl.BlockSpec(a.shape, lambda i: (0,) * a.ndim)
    return pl.pallas_call(
        _tc_body,
        grid=grid,
        in_specs=[
            row(2 * L), row(H), row(H), row(H),
            full(W1t), full(b1t), full(w2t), full(b2t),
            full(W1ci), full(b1ci), full(W2c), full(b2c), full(W2i), full(b2i),
            full(Wa), full(ba),
        ],
        out_specs=[row(1), row(VC), row(VI), row(1)],
        out_shape=[
            jax.ShapeDtypeStruct((N, 1), _F32),
            jax.ShapeDtypeStruct((N, VC), _F32),
            jax.ShapeDtypeStruct((N, VI), _F32),
            jax.ShapeDtypeStruct((N, 1), _F32),
        ],
        compiler_params=pltpu.CompilerParams(
            dimension_semantics=("parallel",),
        ),
    )(ctx, topo_vecs, cls_vecs, assm_vecs,
      W1t, b1t, w2t, b2t, W1ci, b1ci, W2c, b2c, W2i, b2i, Wa, ba)


def kernel(src_tree_vecs, src_graph_vecs, topo_vecs, cls_vecs, assm_vecs,
           batch_idx,
           W1_topo, b1_topo, W2_topo, b2_topo,
           W1_cls, b1_cls, W2_cls, b2_cls,
           W1_icls, b1_icls, W2_icls, b2_icls,
           W_assm, b_assm):
    table = jnp.concatenate([src_tree_vecs, src_graph_vecs], axis=1)
    idx3 = batch_idx.reshape(_NW, _CPW, _CHUNK)
    ctx = _sc_gather(table, idx3)

    W1ci = jnp.concatenate([W1_cls, W1_icls], axis=1)
    b1ci = jnp.concatenate([b1_cls, b1_icls]).reshape(1, 2 * H)
    topo2, cls_s, icls_s, assm2 = _tc_main(
        ctx, topo_vecs.astype(_BF), cls_vecs.astype(_BF), assm_vecs.astype(_BF),
        W1_topo.astype(_BF), b1_topo.reshape(1, H),
        W2_topo.reshape(1, H), b2_topo.reshape(1, 1),
        W1ci.astype(_BF), b1ci,
        W2_cls.astype(_BF), b2_cls.reshape(1, VC),
        W2_icls.astype(_BF), b2_icls.reshape(1, VI),
        W_assm.astype(_BF), b_assm.reshape(1, L),
    )
    return (topo2[:, 0], cls_s, icls_s, assm2[:, 0])


# speedup vs baseline: 1.7212x; 1.0997x over previous
import functools

import jax
import jax.numpy as jnp
from jax import lax
from jax.experimental import pallas as pl
from jax.experimental.pallas import tpu as pltpu
from jax.experimental.pallas import tpu_sc as plsc

N = 160000
B = 4096
H = 128
L = 128
VC = 133
VI = 495

_NW = 32
_CHUNK = 40
_CPW = N // (_NW * _CHUNK)
_D = 2 * L

_NB = 1024

_BF = jnp.bfloat16
_F32 = jnp.float32


def _sc_gather(table, idx3):
    mesh = plsc.VectorSubcoreMesh(core_axis_name="c", subcore_axis_name="s")

    @functools.partial(
        pl.kernel,
        mesh=mesh,
        out_type=jax.ShapeDtypeStruct((N, _D), _F32),
        scratch_types=[
            pltpu.VMEM((_CPW, _CHUNK), jnp.int32),
            pltpu.VMEM((_CHUNK, _D), _F32),
            pltpu.VMEM((_CHUNK, _D), _F32),
            pltpu.SemaphoreType.DMA,
            pltpu.SemaphoreType.DMA,
        ],
    )
    def gather_kernel(table_hbm, idx_hbm, out_hbm, idx_v, buf0, buf1, sem0, sem1):
        wid = lax.axis_index("s") * 2 + lax.axis_index("c")
        cbase = wid * _CPW
        pltpu.sync_copy(idx_hbm.at[wid], idx_v)

        def gat(c, buf, sem):
            return pltpu.make_async_copy(table_hbm.at[idx_v.at[c]], buf, sem)

        def st(c, buf):
            pltpu.sync_copy(buf, out_hbm.at[pl.ds((cbase + c) * _CHUNK, _CHUNK)])

        gat(0, buf0, sem0).start()

        def body(i, carry):
            c0 = 2 * i
            gat(c0, buf0, sem0).wait()
            gat(c0 + 1, buf1, sem1).start()
            st(c0, buf0)
            gat(c0 + 1, buf1, sem1).wait()
            gat(c0 + 2, buf0, sem0).start()
            st(c0 + 1, buf1)
            return carry

        lax.fori_loop(0, (_CPW - 1) // 2, body, 0)
        gat(_CPW - 1, buf0, sem0).wait()
        st(_CPW - 1, buf0)

    return gather_kernel(table, idx3)


def _tc_body(ctx_ref, topo_ref, cls_ref, assm_ref,
             W1t_ref, b1t_ref, w2t_ref, b2t_ref,
             W1ci_ref, b1ci_ref, W2c_ref, b2c_ref, W2i_ref, b2i_ref,
             Wa_ref, ba_ref,
             topo_out, cls_out, icls_out, assm_out):
    tree = ctx_ref[:, :L].astype(_BF)
    graph = ctx_ref[:, L:]
    h_t = jnp.dot(topo_ref[...], W1t_ref[:H], preferred_element_type=_F32)
    h_t = h_t + jnp.dot(tree, W1t_ref[H:], preferred_element_type=_F32)
    h_t = jnp.maximum(h_t + b1t_ref[...], 0.0)
    t = jnp.sum(h_t * w2t_ref[...], axis=1) + b2t_ref[0, 0]
    topo_out[...] = t.reshape(_NB // H, H)
    h_ci = jnp.dot(cls_ref[...], W1ci_ref[:H], preferred_element_type=_F32)
    h_ci = h_ci + jnp.dot(tree, W1ci_ref[H:], preferred_element_type=_F32)
    h_ci = jnp.maximum(h_ci + b1ci_ref[...], 0.0).astype(_BF)
    cls_out[...] = (jnp.dot(h_ci[:, :H], W2c_ref[...], preferred_element_type=_F32)
                    + b2c_ref[...])
    icls_out[...] = (jnp.dot(h_ci[:, H:], W2i_ref[...], preferred_element_type=_F32)
                     + b2i_ref[...])
    a = jnp.dot(assm_ref[...], Wa_ref[...], preferred_element_type=_F32) + ba_ref[...]
    s = jnp.sum(a * graph, axis=1)
    assm_out[...] = s.reshape(_NB // H, H)


def _tc_main(ctx, topo_vecs, cls_vecs, assm_vecs,
             W1t, b1t, w2t, b2t, W1ci, b1ci, W2c, b2c, W2i, b2i, Wa, ba):
    grid = (pl.cdiv(N, _NB),)
    row = lambda w: pl.BlockSpec((_NB, w), lambda i: (i, 0))
    full = lambda a: pl.BlockSpec(a.shape, lambda i: (0,) * a.ndim)
    return pl.pallas_call(
        _tc_body,
        grid=grid,
        in_specs=[
            row(2 * L), row(H), row(H), row(H),
            full(W1t), full(b1t), full(w2t), full(b2t),
            full(W1ci), full(b1ci), full(W2c), full(b2c), full(W2i), full(b2i),
            full(Wa), full(ba),
        ],
        out_specs=[
            pl.BlockSpec((_NB // H, H), lambda i: (i, 0)),
            row(VC), row(VI),
            pl.BlockSpec((_NB // H, H), lambda i: (i, 0)),
        ],
        out_shape=[
            jax.ShapeDtypeStruct((N // H, H), _F32),
            jax.ShapeDtypeStruct((N, VC), _F32),
            jax.ShapeDtypeStruct((N, VI), _F32),
            jax.ShapeDtypeStruct((N // H, H), _F32),
        ],
        compiler_params=pltpu.CompilerParams(
            dimension_semantics=("parallel",),
        ),
    )(ctx, topo_vecs, cls_vecs, assm_vecs,
      W1t, b1t, w2t, b2t, W1ci, b1ci, W2c, b2c, W2i, b2i, Wa, ba)


def kernel(src_tree_vecs, src_graph_vecs, topo_vecs, cls_vecs, assm_vecs,
           batch_idx,
           W1_topo, b1_topo, W2_topo, b2_topo,
           W1_cls, b1_cls, W2_cls, b2_cls,
           W1_icls, b1_icls, W2_icls, b2_icls,
           W_assm, b_assm):
    table = jnp.concatenate([src_tree_vecs, src_graph_vecs], axis=1)
    idx3 = batch_idx.reshape(_NW, _CPW, _CHUNK)
    ctx = _sc_gather(table, idx3)

    W1ci = jnp.concatenate([W1_cls, W1_icls], axis=1)
    b1ci = jnp.concatenate([b1_cls, b1_icls]).reshape(1, 2 * H)
    topo2, cls_s, icls_s, assm2 = _tc_main(
        ctx, topo_vecs.astype(_BF), cls_vecs.astype(_BF), assm_vecs.astype(_BF),
        W1_topo.astype(_BF), b1_topo.reshape(1, H),
        W2_topo.reshape(1, H), b2_topo.reshape(1, 1),
        W1ci.astype(_BF), b1ci,
        W2_cls.astype(_BF), b2_cls.reshape(1, VC),
        W2_icls.astype(_BF), b2_icls.reshape(1, VI),
        W_assm.astype(_BF), b_assm.reshape(1, L),
    )
    return (topo2.reshape(N), cls_s, icls_s, assm2.reshape(N))


# speedup vs baseline: 1.8226x; 1.0589x over previous
import functools

import jax
import jax.numpy as jnp
from jax import lax
from jax.experimental import pallas as pl
from jax.experimental.pallas import tpu as pltpu
from jax.experimental.pallas import tpu_sc as plsc

N = 160000
B = 4096
H = 128
L = 128
VC = 133
VI = 495

_NW = 32
_CHUNK = 40
_CPW = N // (_NW * _CHUNK)
_D = 2 * L

_NB = 1024

_BF = jnp.bfloat16
_F32 = jnp.float32


def _sc_gather(table, idx3):
    mesh = plsc.VectorSubcoreMesh(core_axis_name="c", subcore_axis_name="s")

    @functools.partial(
        pl.kernel,
        mesh=mesh,
        out_type=jax.ShapeDtypeStruct((N, _D), _F32),
        scratch_types=[
            pltpu.VMEM((_CPW, _CHUNK), jnp.int32),
            pltpu.VMEM((_CHUNK, _D), _F32),
            pltpu.VMEM((_CHUNK, _D), _F32),
            pltpu.SemaphoreType.DMA,
            pltpu.SemaphoreType.DMA,
        ],
    )
    def gather_kernel(table_hbm, idx_hbm, out_hbm, idx_v, buf0, buf1, sem0, sem1):
        wid = lax.axis_index("s") * 2 + lax.axis_index("c")
        cbase = wid * _CPW
        pltpu.sync_copy(idx_hbm.at[wid], idx_v)

        def gat(c, buf, sem):
            return pltpu.make_async_copy(table_hbm.at[idx_v.at[c]], buf, sem)

        def st(c, buf):
            pltpu.sync_copy(buf, out_hbm.at[pl.ds((cbase + c) * _CHUNK, _CHUNK)])

        gat(0, buf0, sem0).start()

        def body(i, carry):
            c0 = 2 * i
            gat(c0, buf0, sem0).wait()
            gat(c0 + 1, buf1, sem1).start()
            st(c0, buf0)
            gat(c0 + 1, buf1, sem1).wait()
            gat(c0 + 2, buf0, sem0).start()
            st(c0 + 1, buf1)
            return carry

        lax.fori_loop(0, (_CPW - 1) // 2, body, 0)
        gat(_CPW - 1, buf0, sem0).wait()
        st(_CPW - 1, buf0)

    return gather_kernel(table, idx3)


def _tc_body(ctx_ref, topo_ref, cls_ref, assm_ref,
             W1t_ref, b1t_ref, w2t_ref, b2t_ref,
             W1ci_ref, b1ci_ref, W2c_ref, b2c_ref, W2i_ref, b2i_ref,
             Wa_ref, ba_ref,
             topo_out, cls_out, icls_out, assm_out):
    tree = ctx_ref[:, :L].astype(_BF)
    graph = ctx_ref[:, L:]
    h_t = jnp.dot(topo_ref[...].astype(_BF), W1t_ref[:H], preferred_element_type=_F32)
    h_t = h_t + jnp.dot(tree, W1t_ref[H:], preferred_element_type=_F32)
    h_t = jnp.maximum(h_t + b1t_ref[...], 0.0)
    t = jnp.sum(h_t * w2t_ref[...], axis=1) + b2t_ref[0, 0]
    topo_out[...] = t.reshape(_NB // H, H)
    h_ci = jnp.dot(cls_ref[...].astype(_BF), W1ci_ref[:H], preferred_element_type=_F32)
    h_ci = h_ci + jnp.dot(tree, W1ci_ref[H:], preferred_element_type=_F32)
    h_ci = jnp.maximum(h_ci + b1ci_ref[...], 0.0).astype(_BF)
    cls_out[...] = (jnp.dot(h_ci[:, :H], W2c_ref[...], preferred_element_type=_F32)
                    + b2c_ref[...])
    icls_out[...] = (jnp.dot(h_ci[:, H:], W2i_ref[...], preferred_element_type=_F32)
                     + b2i_ref[...])
    a = jnp.dot(assm_ref[...].astype(_BF), Wa_ref[...], preferred_element_type=_F32) + ba_ref[...]
    s = jnp.sum(a * graph, axis=1)
    assm_out[...] = s.reshape(_NB // H, H)


def _tc_main(ctx, topo_vecs, cls_vecs, assm_vecs,
             W1t, b1t, w2t, b2t, W1ci, b1ci, W2c, b2c, W2i, b2i, Wa, ba):
    grid = (pl.cdiv(N, _NB),)
    row = lambda w: pl.BlockSpec((_NB, w), lambda i: (i, 0))
    full = lambda a: pl.BlockSpec(a.shape, lambda i: (0,) * a.ndim)
    return pl.pallas_call(
        _tc_body,
        grid=grid,
        in_specs=[
            row(2 * L), row(H), row(H), row(H),
            full(W1t), full(b1t), full(w2t), full(b2t),
            full(W1ci), full(b1ci), full(W2c), full(b2c), full(W2i), full(b2i),
            full(Wa), full(ba),
        ],
        out_specs=[
            pl.BlockSpec((_NB // H, H), lambda i: (i, 0)),
            row(VC), row(VI),
            pl.BlockSpec((_NB // H, H), lambda i: (i, 0)),
        ],
        out_shape=[
            jax.ShapeDtypeStruct((N // H, H), _F32),
            jax.ShapeDtypeStruct((N, VC), _F32),
            jax.ShapeDtypeStruct((N, VI), _F32),
            jax.ShapeDtypeStruct((N // H, H), _F32),
        ],
        compiler_params=pltpu.CompilerParams(
            dimension_semantics=("parallel",),
        ),
    )(ctx, topo_vecs, cls_vecs, assm_vecs,
      W1t, b1t, w2t, b2t, W1ci, b1ci, W2c, b2c, W2i, b2i, Wa, ba)


def kernel(src_tree_vecs, src_graph_vecs, topo_vecs, cls_vecs, assm_vecs,
           batch_idx,
           W1_topo, b1_topo, W2_topo, b2_topo,
           W1_cls, b1_cls, W2_cls, b2_cls,
           W1_icls, b1_icls, W2_icls, b2_icls,
           W_assm, b_assm):
    table = jnp.concatenate([src_tree_vecs, src_graph_vecs], axis=1)
    idx3 = batch_idx.reshape(_NW, _CPW, _CHUNK)
    ctx = _sc_gather(table, idx3)

    W1ci = jnp.concatenate([W1_cls, W1_icls], axis=1)
    b1ci = jnp.concatenate([b1_cls, b1_icls]).reshape(1, 2 * H)
    topo2, cls_s, icls_s, assm2 = _tc_main(
        ctx, topo_vecs, cls_vecs, assm_vecs,
        W1_topo.astype(_BF), b1_topo.reshape(1, H),
        W2_topo.reshape(1, H), b2_topo.reshape(1, 1),
        W1ci.astype(_BF), b1ci,
        W2_cls.astype(_BF), b2_cls.reshape(1, VC),
        W2_icls.astype(_BF), b2_icls.reshape(1, VI),
        W_assm.astype(_BF), b_assm.reshape(1, L),
    )
    return (topo2.reshape(N), cls_s, icls_s, assm2.reshape(N))


# speedup vs baseline: 1.9461x; 1.0678x over previous
import functools

import jax
import jax.numpy as jnp
from jax import lax
from jax.experimental import pallas as pl
from jax.experimental.pallas import tpu as pltpu
from jax.experimental.pallas import tpu_sc as plsc

N = 160000
B = 4096
H = 128
L = 128
VC = 133
VI = 495

_NW = 32
_CHUNK = 40
_CPW = N // (_NW * _CHUNK)
_D = L

_NB = 1024

_BF = jnp.bfloat16
_F32 = jnp.float32


def _sc_gather(table, idx3):
    mesh = plsc.VectorSubcoreMesh(core_axis_name="c", subcore_axis_name="s")

    @functools.partial(
        pl.kernel,
        mesh=mesh,
        out_type=jax.ShapeDtypeStruct((N, _D), jnp.int32),
        scratch_types=[
            pltpu.VMEM((_CPW, _CHUNK), jnp.int32),
            pltpu.VMEM((_CHUNK, _D), jnp.int32),
            pltpu.VMEM((_CHUNK, _D), jnp.int32),
            pltpu.SemaphoreType.DMA,
            pltpu.SemaphoreType.DMA,
        ],
    )
    def gather_kernel(table_hbm, idx_hbm, out_hbm, idx_v, buf0, buf1, sem0, sem1):
        wid = lax.axis_index("s") * 2 + lax.axis_index("c")
        cbase = wid * _CPW
        pltpu.sync_copy(idx_hbm.at[wid], idx_v)

        def gat(c, buf, sem):
            return pltpu.make_async_copy(table_hbm.at[idx_v.at[c]], buf, sem)

        def st(c, buf):
            pltpu.sync_copy(buf, out_hbm.at[pl.ds((cbase + c) * _CHUNK, _CHUNK)])

        gat(0, buf0, sem0).start()

        def body(i, carry):
            c0 = 2 * i
            gat(c0, buf0, sem0).wait()
            gat(c0 + 1, buf1, sem1).start()
            st(c0, buf0)
            gat(c0 + 1, buf1, sem1).wait()
            gat(c0 + 2, buf0, sem0).start()
            st(c0 + 1, buf1)
            return carry

        lax.fori_loop(0, (_CPW - 1) // 2, body, 0)
        gat(_CPW - 1, buf0, sem0).wait()
        st(_CPW - 1, buf0)

    return gather_kernel(table, idx3)


def _tc_body(ctx_ref, topo_ref, cls_ref, assm_ref,
             W1t_ref, b1t_ref, w2t_ref, b2t_ref,
             W1ci_ref, b1ci_ref, W2c_ref, b2c_ref, W2i_ref, b2i_ref,
             Wa_ref, ba_ref,
             topo_out, cls_out, icls_out, assm_out):
    ctx_u = lax.bitcast_convert_type(ctx_ref[...], jnp.uint32)
    tree_f = lax.bitcast_convert_type(ctx_u << 16, _F32)
    graph = lax.bitcast_convert_type(ctx_u & jnp.uint32(0xFFFF0000), _F32)
    tree = tree_f.astype(_BF)
    h_t = jnp.dot(topo_ref[...].astype(_BF), W1t_ref[:H], preferred_element_type=_F32)
    h_t = h_t + jnp.dot(tree, W1t_ref[H:], preferred_element_type=_F32)
    h_t = jnp.maximum(h_t + b1t_ref[...], 0.0)
    t = jnp.sum(h_t * w2t_ref[...], axis=1) + b2t_ref[0, 0]
    topo_out[...] = t.reshape(_NB // H, H)
    h_ci = jnp.dot(cls_ref[...].astype(_BF), W1ci_ref[:H], preferred_element_type=_F32)
    h_ci = h_ci + jnp.dot(tree, W1ci_ref[H:], preferred_element_type=_F32)
    h_ci = jnp.maximum(h_ci + b1ci_ref[...], 0.0).astype(_BF)
    cls_out[...] = (jnp.dot(h_ci[:, :H], W2c_ref[...], preferred_element_type=_F32)
                    + b2c_ref[...])
    icls_out[...] = (jnp.dot(h_ci[:, H:], W2i_ref[...], preferred_element_type=_F32)
                     + b2i_ref[...])
    a = jnp.dot(assm_ref[...].astype(_BF), Wa_ref[...], preferred_element_type=_F32) + ba_ref[...]
    s = jnp.sum(a * graph, axis=1)
    assm_out[...] = s.reshape(_NB // H, H)


def _tc_main(ctx, topo_vecs, cls_vecs, assm_vecs,
             W1t, b1t, w2t, b2t, W1ci, b1ci, W2c, b2c, W2i, b2i, Wa, ba):
    grid = (pl.cdiv(N, _NB),)
    row = lambda w: pl.BlockSpec((_NB, w), lambda i: (i, 0))
    full = lambda a: pl.BlockSpec(a.shape, lambda i: (0,) * a.ndim)
    return pl.pallas_call(
        _tc_body,
        grid=grid,
        in_specs=[
            row(_D), row(H), row(H), row(H),
            full(W1t), full(b1t), full(w2t), full(b2t),
            full(W1ci), full(b1ci), full(W2c), full(b2c), full(W2i), full(b2i),
            full(Wa), full(ba),
        ],
        out_specs=[
            pl.BlockSpec((_NB // H, H), lambda i: (i, 0)),
            row(VC), row(VI),
            pl.BlockSpec((_NB // H, H), lambda i: (i, 0)),
        ],
        out_shape=[
            jax.ShapeDtypeStruct((N // H, H), _F32),
            jax.ShapeDtypeStruct((N, VC), _F32),
            jax.ShapeDtypeStruct((N, VI), _F32),
            jax.ShapeDtypeStruct((N // H, H), _F32),
        ],
        compiler_params=pltpu.CompilerParams(
            dimension_semantics=("parallel",),
        ),
    )(ctx, topo_vecs, cls_vecs, assm_vecs,
      W1t, b1t, w2t, b2t, W1ci, b1ci, W2c, b2c, W2i, b2i, Wa, ba)


def kernel(src_tree_vecs, src_graph_vecs, topo_vecs, cls_vecs, assm_vecs,
           batch_idx,
           W1_topo, b1_topo, W2_topo, b2_topo,
           W1_cls, b1_cls, W2_cls, b2_cls,
           W1_icls, b1_icls, W2_icls, b2_icls,
           W_assm, b_assm):
    t16 = lax.bitcast_convert_type(src_tree_vecs.astype(_BF), jnp.uint16)
    g16 = lax.bitcast_convert_type(src_graph_vecs.astype(_BF), jnp.uint16)
    table = lax.bitcast_convert_type(
        (g16.astype(jnp.uint32) << 16) | t16.astype(jnp.uint32), jnp.int32)
    idx3 = batch_idx.reshape(_NW, _CPW, _CHUNK)
    ctx = _sc_gather(table, idx3)

    W1ci = jnp.concatenate([W1_cls, W1_icls], axis=1)
    b1ci = jnp.concatenate([b1_cls, b1_icls]).reshape(1, 2 * H)
    topo2, cls_s, icls_s, assm2 = _tc_main(
        ctx, topo_vecs, cls_vecs, assm_vecs,
        W1_topo.astype(_BF), b1_topo.reshape(1, H),
        W2_topo.reshape(1, H), b2_topo.reshape(1, 1),
        W1ci.astype(_BF), b1ci,
        W2_cls.astype(_BF), b2_cls.reshape(1, VC),
        W2_icls.astype(_BF), b2_icls.reshape(1, VI),
        W_assm.astype(_BF), b_assm.reshape(1, L),
    )
    return (topo2.reshape(N), cls_s, icls_s, assm2.reshape(N))


# speedup vs baseline: 2.0743x; 1.0659x over previous
import functools

import jax
import jax.numpy as jnp
from jax import lax
from jax.experimental import pallas as pl
from jax.experimental.pallas import tpu as pltpu
from jax.experimental.pallas import tpu_sc as plsc

N = 160000
B = 4096
H = 128
L = 128
VC = 133
VI = 495

_NW = 32
_CHUNK = 40
_CPW = N // (_NW * _CHUNK)
_D = L

_NB = 2048

_BF = jnp.bfloat16
_F32 = jnp.float32


def _sc_gather(table, idx3):
    mesh = plsc.VectorSubcoreMesh(core_axis_name="c", subcore_axis_name="s")

    @functools.partial(
        pl.kernel,
        mesh=mesh,
        out_type=jax.ShapeDtypeStruct((N, _D), jnp.int32),
        scratch_types=[
            pltpu.VMEM((_CPW, _CHUNK), jnp.int32),
            pltpu.VMEM((_CHUNK, _D), jnp.int32),
            pltpu.VMEM((_CHUNK, _D), jnp.int32),
            pltpu.SemaphoreType.DMA,
            pltpu.SemaphoreType.DMA,
        ],
    )
    def gather_kernel(table_hbm, idx_hbm, out_hbm, idx_v, buf0, buf1, sem0, sem1):
        wid = lax.axis_index("s") * 2 + lax.axis_index("c")
        cbase = wid * _CPW
        pltpu.sync_copy(idx_hbm.at[wid], idx_v)

        def gat(c, buf, sem):
            return pltpu.make_async_copy(table_hbm.at[idx_v.at[c]], buf, sem)

        def st(c, buf):
            pltpu.sync_copy(buf, out_hbm.at[pl.ds((cbase + c) * _CHUNK, _CHUNK)])

        gat(0, buf0, sem0).start()

        def body(i, carry):
            c0 = 2 * i
            gat(c0, buf0, sem0).wait()
            gat(c0 + 1, buf1, sem1).start()
            st(c0, buf0)
            gat(c0 + 1, buf1, sem1).wait()
            gat(c0 + 2, buf0, sem0).start()
            st(c0 + 1, buf1)
            return carry

        lax.fori_loop(0, (_CPW - 1) // 2, body, 0)
        gat(_CPW - 1, buf0, sem0).wait()
        st(_CPW - 1, buf0)

    return gather_kernel(table, idx3)


def _tc_body(ctx_ref, topo_ref, cls_ref, assm_ref,
             W1t_ref, b1t_ref, w2t_ref, b2t_ref,
             W1ci_ref, b1ci_ref, W2c_ref, b2c_ref, W2i_ref, b2i_ref,
             Wa_ref, ba_ref,
             topo_out, cls_out, icls_out, assm_out):
    ctx_u = lax.bitcast_convert_type(ctx_ref[...], jnp.uint32)
    tree_f = lax.bitcast_convert_type(ctx_u << 16, _F32)
    graph = lax.bitcast_convert_type(ctx_u & jnp.uint32(0xFFFF0000), _F32)
    tree = tree_f.astype(_BF)
    h_t = jnp.dot(topo_ref[...].astype(_BF), W1t_ref[:H], preferred_element_type=_F32)
    h_t = h_t + jnp.dot(tree, W1t_ref[H:], preferred_element_type=_F32)
    h_t = jnp.maximum(h_t + b1t_ref[...], 0.0)
    t = jnp.sum(h_t * w2t_ref[...], axis=1) + b2t_ref[0, 0]
    topo_out[...] = t.reshape(_NB // H, H)
    h_ci = jnp.dot(cls_ref[...].astype(_BF), W1ci_ref[:H], preferred_element_type=_F32)
    h_ci = h_ci + jnp.dot(tree, W1ci_ref[H:], preferred_element_type=_F32)
    h_ci = jnp.maximum(h_ci + b1ci_ref[...], 0.0).astype(_BF)
    cls_out[...] = (jnp.dot(h_ci[:, :H], W2c_ref[...], preferred_element_type=_F32)
                    + b2c_ref[...])
    icls_out[...] = (jnp.dot(h_ci[:, H:], W2i_ref[...], preferred_element_type=_F32)
                     + b2i_ref[...])
    a = jnp.dot(assm_ref[...].astype(_BF), Wa_ref[...], preferred_element_type=_F32) + ba_ref[...]
    s = jnp.sum(a * graph, axis=1)
    assm_out[...] = s.reshape(_NB // H, H)


def _tc_main(ctx, topo_vecs, cls_vecs, assm_vecs,
             W1t, b1t, w2t, b2t, W1ci, b1ci, W2c, b2c, W2i, b2i, Wa, ba):
    grid = (pl.cdiv(N, _NB),)
    row = lambda w: pl.BlockSpec((_NB, w), lambda i: (i, 0))
    full = lambda a: pl.BlockSpec(a.shape, lambda i: (0,) * a.ndim)
    return pl.pallas_call(
        _tc_body,
        grid=grid,
        in_specs=[
            row(_D), row(H), row(H), row(H),
            full(W1t), full(b1t), full(w2t), full(b2t),
            full(W1ci), full(b1ci), full(W2c), full(b2c), full(W2i), full(b2i),
            full(Wa), full(ba),
        ],
        out_specs=[
            pl.BlockSpec((_NB // H, H), lambda i: (i, 0)),
            row(VC), row(VI),
            pl.BlockSpec((_NB // H, H), lambda i: (i, 0)),
        ],
        out_shape=[
            jax.ShapeDtypeStruct((N // H, H), _F32),
            jax.ShapeDtypeStruct((N, VC), _F32),
            jax.ShapeDtypeStruct((N, VI), _F32),
            jax.ShapeDtypeStruct((N // H, H), _F32),
        ],
        compiler_params=pltpu.CompilerParams(
            dimension_semantics=("parallel",),
        ),
    )(ctx, topo_vecs, cls_vecs, assm_vecs,
      W1t, b1t, w2t, b2t, W1ci, b1ci, W2c, b2c, W2i, b2i, Wa, ba)


def kernel(src_tree_vecs, src_graph_vecs, topo_vecs, cls_vecs, assm_vecs,
           batch_idx,
           W1_topo, b1_topo, W2_topo, b2_topo,
           W1_cls, b1_cls, W2_cls, b2_cls,
           W1_icls, b1_icls, W2_icls, b2_icls,
           W_assm, b_assm):
    t16 = lax.bitcast_convert_type(src_tree_vecs.astype(_BF), jnp.uint16)
    g16 = lax.bitcast_convert_type(src_graph_vecs.astype(_BF), jnp.uint16)
    table = lax.bitcast_convert_type(
        (g16.astype(jnp.uint32) << 16) | t16.astype(jnp.uint32), jnp.int32)
    idx3 = batch_idx.reshape(_NW, _CPW, _CHUNK)
    ctx = _sc_gather(table, idx3)

    W1ci = jnp.concatenate([W1_cls, W1_icls], axis=1)
    b1ci = jnp.concatenate([b1_cls, b1_icls]).reshape(1, 2 * H)
    topo2, cls_s, icls_s, assm2 = _tc_main(
        ctx, topo_vecs, cls_vecs, assm_vecs,
        W1_topo.astype(_BF), b1_topo.reshape(1, H),
        W2_topo.reshape(1, H), b2_topo.reshape(1, 1),
        W1ci.astype(_BF), b1ci,
        W2_cls.astype(_BF), b2_cls.reshape(1, VC),
        W2_icls.astype(_BF), b2_icls.reshape(1, VI),
        W_assm.astype(_BF), b_assm.reshape(1, L),
    )
    return (topo2.reshape(N), cls_s, icls_s, assm2.reshape(N))


# speedup vs baseline: 2.1239x; 1.0239x over previous
import functools

import jax
import jax.numpy as jnp
from jax import lax
from jax.experimental import pallas as pl
from jax.experimental.pallas import tpu as pltpu
from jax.experimental.pallas import tpu_sc as plsc

N = 160000
B = 4096
H = 128
L = 128
VC = 133
VI = 495

_NW = 32
_CHUNK = 40
_CPW = N // (_NW * _CHUNK)
_D = L

_NB = 4096

_BF = jnp.bfloat16
_F32 = jnp.float32


def _sc_gather(table, idx3):
    mesh = plsc.VectorSubcoreMesh(core_axis_name="c", subcore_axis_name="s")

    @functools.partial(
        pl.kernel,
        mesh=mesh,
        out_type=jax.ShapeDtypeStruct((N, _D), jnp.int32),
        scratch_types=[
            pltpu.VMEM((_CPW, _CHUNK), jnp.int32),
            pltpu.VMEM((_CHUNK, _D), jnp.int32),
            pltpu.VMEM((_CHUNK, _D), jnp.int32),
            pltpu.SemaphoreType.DMA,
            pltpu.SemaphoreType.DMA,
        ],
    )
    def gather_kernel(table_hbm, idx_hbm, out_hbm, idx_v, buf0, buf1, sem0, sem1):
        wid = lax.axis_index("s") * 2 + lax.axis_index("c")
        cbase = wid * _CPW
        pltpu.sync_copy(idx_hbm.at[wid], idx_v)

        def gat(c, buf, sem):
            return pltpu.make_async_copy(table_hbm.at[idx_v.at[c]], buf, sem)

        def st(c, buf):
            pltpu.sync_copy(buf, out_hbm.at[pl.ds((cbase + c) * _CHUNK, _CHUNK)])

        gat(0, buf0, sem0).start()

        def body(i, carry):
            c0 = 2 * i
            gat(c0, buf0, sem0).wait()
            gat(c0 + 1, buf1, sem1).start()
            st(c0, buf0)
            gat(c0 + 1, buf1, sem1).wait()
            gat(c0 + 2, buf0, sem0).start()
            st(c0 + 1, buf1)
            return carry

        lax.fori_loop(0, (_CPW - 1) // 2, body, 0)
        gat(_CPW - 1, buf0, sem0).wait()
        st(_CPW - 1, buf0)

    return gather_kernel(table, idx3)


def _tc_body(ctx_ref, topo_ref, cls_ref, assm_ref,
             W1t_ref, b1t_ref, w2t_ref, b2t_ref,
             W1ci_ref, b1ci_ref, W2c_ref, b2c_ref, W2i_ref, b2i_ref,
             Wa_ref, ba_ref,
             topo_out, cls_out, icls_out, assm_out):
    ctx_u = lax.bitcast_convert_type(ctx_ref[...], jnp.uint32)
    tree_f = lax.bitcast_convert_type(ctx_u << 16, _F32)
    graph = lax.bitcast_convert_type(ctx_u & jnp.uint32(0xFFFF0000), _F32)
    tree = tree_f.astype(_BF)
    h_t = jnp.dot(topo_ref[...].astype(_BF), W1t_ref[:H], preferred_element_type=_F32)
    h_t = h_t + jnp.dot(tree, W1t_ref[H:], preferred_element_type=_F32)
    h_t = jnp.maximum(h_t + b1t_ref[...], 0.0)
    t = jnp.sum(h_t * w2t_ref[...], axis=1) + b2t_ref[0, 0]
    topo_out[...] = t.reshape(_NB // H, H)
    h_ci = jnp.dot(cls_ref[...].astype(_BF), W1ci_ref[:H], preferred_element_type=_F32)
    h_ci = h_ci + jnp.dot(tree, W1ci_ref[H:], preferred_element_type=_F32)
    h_ci = jnp.maximum(h_ci + b1ci_ref[...], 0.0).astype(_BF)
    cls_out[...] = (jnp.dot(h_ci[:, :H], W2c_ref[...], preferred_element_type=_F32)
                    + b2c_ref[...])
    icls_out[...] = (jnp.dot(h_ci[:, H:], W2i_ref[...], preferred_element_type=_F32)
                     + b2i_ref[...])
    a = jnp.dot(assm_ref[...].astype(_BF), Wa_ref[...], preferred_element_type=_F32) + ba_ref[...]
    s = jnp.sum(a * graph, axis=1)
    assm_out[...] = s.reshape(_NB // H, H)


def _tc_main(ctx, topo_vecs, cls_vecs, assm_vecs,
             W1t, b1t, w2t, b2t, W1ci, b1ci, W2c, b2c, W2i, b2i, Wa, ba):
    grid = (pl.cdiv(N, _NB),)
    row = lambda w: pl.BlockSpec((_NB, w), lambda i: (i, 0))
    full = lambda a: pl.BlockSpec(a.shape, lambda i: (0,) * a.ndim)
    return pl.pallas_call(
        _tc_body,
        grid=grid,
        in_specs=[
            row(_D), row(H), row(H), row(H),
            full(W1t), full(b1t), full(w2t), full(b2t),
            full(W1ci), full(b1ci), full(W2c), full(b2c), full(W2i), full(b2i),
            full(Wa), full(ba),
        ],
        out_specs=[
            pl.BlockSpec((_NB // H, H), lambda i: (i, 0)),
            row(VC), row(VI),
            pl.BlockSpec((_NB // H, H), lambda i: (i, 0)),
        ],
        out_shape=[
            jax.ShapeDtypeStruct((N // H, H), _F32),
            jax.ShapeDtypeStruct((N, VC), _F32),
            jax.ShapeDtypeStruct((N, VI), _F32),
            jax.ShapeDtypeStruct((N // H, H), _F32),
        ],
        compiler_params=pltpu.CompilerParams(
            dimension_semantics=("parallel",),
        ),
    )(ctx, topo_vecs, cls_vecs, assm_vecs,
      W1t, b1t, w2t, b2t, W1ci, b1ci, W2c, b2c, W2i, b2i, Wa, ba)


def kernel(src_tree_vecs, src_graph_vecs, topo_vecs, cls_vecs, assm_vecs,
           batch_idx,
           W1_topo, b1_topo, W2_topo, b2_topo,
           W1_cls, b1_cls, W2_cls, b2_cls,
           W1_icls, b1_icls, W2_icls, b2_icls,
           W_assm, b_assm):
    t16 = lax.bitcast_convert_type(src_tree_vecs.astype(_BF), jnp.uint16)
    g16 = lax.bitcast_convert_type(src_graph_vecs.astype(_BF), jnp.uint16)
    table = lax.bitcast_convert_type(
        (g16.astype(jnp.uint32) << 16) | t16.astype(jnp.uint32), jnp.int32)
    idx3 = batch_idx.reshape(_NW, _CPW, _CHUNK)
    ctx = _sc_gather(table, idx3)

    W1ci = jnp.concatenate([W1_cls, W1_icls], axis=1)
    b1ci = jnp.concatenate([b1_cls, b1_icls]).reshape(1, 2 * H)
    topo2, cls_s, icls_s, assm2 = _tc_main(
        ctx, topo_vecs, cls_vecs, assm_vecs,
        W1_topo.astype(_BF), b1_topo.reshape(1, H),
        W2_topo.reshape(1, H), b2_topo.reshape(1, 1),
        W1ci.astype(_BF), b1ci,
        W2_cls.astype(_BF), b2_cls.reshape(1, VC),
        W2_icls.astype(_BF), b2_icls.reshape(1, VI),
        W_assm.astype(_BF), b_assm.reshape(1, L),
    )
    return (topo2.reshape(N), cls_s, icls_s, assm2.reshape(N))


# speedup vs baseline: 2.1277x; 1.0018x over previous
import functools

import jax
import jax.numpy as jnp
from jax import lax
from jax.experimental import pallas as pl
from jax.experimental.pallas import tpu as pltpu
from jax.experimental.pallas import tpu_sc as plsc

N = 160000
B = 4096
H = 128
L = 128
VC = 133
VI = 495

_NW = 32
_CHUNK = 40
_CPW = N // (_NW * _CHUNK)
_D = L

_NB = 5120

_BF = jnp.bfloat16
_F32 = jnp.float32


def _sc_gather(table, idx3):
    mesh = plsc.VectorSubcoreMesh(core_axis_name="c", subcore_axis_name="s")

    @functools.partial(
        pl.kernel,
        mesh=mesh,
        out_type=jax.ShapeDtypeStruct((N, _D), jnp.int32),
        scratch_types=[
            pltpu.VMEM((_CPW, _CHUNK), jnp.int32),
            pltpu.VMEM((_CHUNK, _D), jnp.int32),
            pltpu.VMEM((_CHUNK, _D), jnp.int32),
            pltpu.SemaphoreType.DMA,
            pltpu.SemaphoreType.DMA,
        ],
    )
    def gather_kernel(table_hbm, idx_hbm, out_hbm, idx_v, buf0, buf1, sem0, sem1):
        wid = lax.axis_index("s") * 2 + lax.axis_index("c")
        cbase = wid * _CPW
        pltpu.sync_copy(idx_hbm.at[wid], idx_v)

        def gat(c, buf, sem):
            return pltpu.make_async_copy(table_hbm.at[idx_v.at[c]], buf, sem)

        def st(c, buf):
            pltpu.sync_copy(buf, out_hbm.at[pl.ds((cbase + c) * _CHUNK, _CHUNK)])

        gat(0, buf0, sem0).start()

        def body(i, carry):
            c0 = 2 * i
            gat(c0, buf0, sem0).wait()
            gat(c0 + 1, buf1, sem1).start()
            st(c0, buf0)
            gat(c0 + 1, buf1, sem1).wait()
            gat(c0 + 2, buf0, sem0).start()
            st(c0 + 1, buf1)
            return carry

        lax.fori_loop(0, (_CPW - 1) // 2, body, 0)
        gat(_CPW - 1, buf0, sem0).wait()
        st(_CPW - 1, buf0)

    return gather_kernel(table, idx3)


def _tc_body(ctx_ref, topo_ref, cls_ref, assm_ref,
             W1t_ref, b1t_ref, w2t_ref, b2t_ref,
             W1ci_ref, b1ci_ref, W2c_ref, b2c_ref, W2i_ref, b2i_ref,
             Wa_ref, ba_ref,
             topo_out, cls_out, icls_out, assm_out):
    ctx_u = lax.bitcast_convert_type(ctx_ref[...], jnp.uint32)
    tree_f = lax.bitcast_convert_type(ctx_u << 16, _F32)
    graph = lax.bitcast_convert_type(ctx_u & jnp.uint32(0xFFFF0000), _F32)
    tree = tree_f.astype(_BF)
    h_t = jnp.dot(topo_ref[...].astype(_BF), W1t_ref[:H], preferred_element_type=_F32)
    h_t = h_t + jnp.dot(tree, W1t_ref[H:], preferred_element_type=_F32)
    h_t = jnp.maximum(h_t + b1t_ref[...], 0.0)
    t = jnp.sum(h_t * w2t_ref[...], axis=1) + b2t_ref[0, 0]
    topo_out[...] = t.reshape(_NB // H, H)
    h_ci = jnp.dot(cls_ref[...].astype(_BF), W1ci_ref[:H], preferred_element_type=_F32)
    h_ci = h_ci + jnp.dot(tree, W1ci_ref[H:], preferred_element_type=_F32)
    h_ci = jnp.maximum(h_ci + b1ci_ref[...], 0.0).astype(_BF)
    cls_out[...] = (jnp.dot(h_ci[:, :H], W2c_ref[...], preferred_element_type=_F32)
                    + b2c_ref[...])
    icls_out[...] = (jnp.dot(h_ci[:, H:], W2i_ref[...], preferred_element_type=_F32)
                     + b2i_ref[...])
    a = jnp.dot(assm_ref[...].astype(_BF), Wa_ref[...], preferred_element_type=_F32) + ba_ref[...]
    s = jnp.sum(a * graph, axis=1)
    assm_out[...] = s.reshape(_NB // H, H)


def _tc_main(ctx, topo_vecs, cls_vecs, assm_vecs,
             W1t, b1t, w2t, b2t, W1ci, b1ci, W2c, b2c, W2i, b2i, Wa, ba):
    grid = (pl.cdiv(N, _NB),)
    row = lambda w: pl.BlockSpec((_NB, w), lambda i: (i, 0))
    full = lambda a: pl.BlockSpec(a.shape, lambda i: (0,) * a.ndim)
    return pl.pallas_call(
        _tc_body,
        grid=grid,
        in_specs=[
            row(_D), row(H), row(H), row(H),
            full(W1t), full(b1t), full(w2t), full(b2t),
            full(W1ci), full(b1ci), full(W2c), full(b2c), full(W2i), full(b2i),
            full(Wa), full(ba),
        ],
        out_specs=[
            pl.BlockSpec((_NB // H, H), lambda i: (i, 0)),
            row(VC), row(VI),
            pl.BlockSpec((_NB // H, H), lambda i: (i, 0)),
        ],
        out_shape=[
            jax.ShapeDtypeStruct((N // H, H), _F32),
            jax.ShapeDtypeStruct((N, VC), _F32),
            jax.ShapeDtypeStruct((N, VI), _F32),
            jax.ShapeDtypeStruct((N // H, H), _F32),
        ],
        compiler_params=pltpu.CompilerParams(
            dimension_semantics=("parallel",),
        ),
    )(ctx, topo_vecs, cls_vecs, assm_vecs,
      W1t, b1t, w2t, b2t, W1ci, b1ci, W2c, b2c, W2i, b2i, Wa, ba)


def kernel(src_tree_vecs, src_graph_vecs, topo_vecs, cls_vecs, assm_vecs,
           batch_idx,
           W1_topo, b1_topo, W2_topo, b2_topo,
           W1_cls, b1_cls, W2_cls, b2_cls,
           W1_icls, b1_icls, W2_icls, b2_icls,
           W_assm, b_assm):
    t16 = lax.bitcast_convert_type(src_tree_vecs.astype(_BF), jnp.uint16)
    g16 = lax.bitcast_convert_type(src_graph_vecs.astype(_BF), jnp.uint16)
    table = lax.bitcast_convert_type(
        (g16.astype(jnp.uint32) << 16) | t16.astype(jnp.uint32), jnp.int32)
    idx3 = batch_idx.reshape(_NW, _CPW, _CHUNK)
    ctx = _sc_gather(table, idx3)

    W1ci = jnp.concatenate([W1_cls, W1_icls], axis=1)
    b1ci = jnp.concatenate([b1_cls, b1_icls]).reshape(1, 2 * H)
    topo2, cls_s, icls_s, assm2 = _tc_main(
        ctx, topo_vecs, cls_vecs, assm_vecs,
        W1_topo.astype(_BF), b1_topo.reshape(1, H),
        W2_topo.reshape(1, H), b2_topo.reshape(1, 1),
        W1ci.astype(_BF), b1ci,
        W2_cls.astype(_BF), b2_cls.reshape(1, VC),
        W2_icls.astype(_BF), b2_icls.reshape(1, VI),
        W_assm.astype(_BF), b_assm.reshape(1, L),
    )
    return (topo2.reshape(N), cls_s, icls_s, assm2.reshape(N))
